# baseline (speedup 1.0000x reference)
import functools

import jax
import jax.numpy as jnp
from jax import lax
from jax.experimental import pallas as pl
from jax.experimental.pallas import tpu as pltpu
from jax.experimental.pallas import tpu_sc as plsc

NUM_NODES = 1_000_000
EMBED_DIM = 64
BATCH = 16384
HIST = 50

_info = plsc.get_sparse_core_info()
NC, NS = _info.num_cores, _info.num_subcores
NW = NC * NS
B_PER_W = BATCH // NW
C = 256
QC = B_PER_W // C
T_STEPS = HIST * QC


W = 512
NSLAB = (NUM_NODES // W)
TAIL0 = NSLAB * W


def _repack(tableT):
    mesh = plsc.VectorSubcoreMesh(core_axis_name="c", subcore_axis_name="s")

    @functools.partial(
        pl.kernel,
        mesh=mesh,
        out_type=jax.ShapeDtypeStruct((NUM_NODES // 2, 2 * EMBED_DIM),
                                      jnp.float32),
        scratch_types=[
            pltpu.VMEM((EMBED_DIM, W), jnp.float32),
            pltpu.VMEM((W // 2, 2 * EMBED_DIM), jnp.float32),
            pltpu.VMEM((EMBED_DIM, EMBED_DIM), jnp.float32),
        ],
        compiler_params=pltpu.CompilerParams(use_tc_tiling_on_sc=True,
                                             needs_layout_passes=False),
    )
    def body(tableT_hbm, tableP_hbm, slab_v, out_v, tail_v):
        wid = lax.axis_index("s") * NC + lax.axis_index("c")
        lane = lax.iota(jnp.int32, 16)

        def transpose_blocks(src, n_rows):
            def blk(j0g, carry):
                row_ids = j0g * 16 + lane
                jj2 = 2 * row_ids
                for k in range(16):
                    dk = (lane + k) & 15
                    for d0 in range(0, EMBED_DIM, 16):
                        d_ids = d0 + dk
                        v0 = plsc.load_gather(src, [d_ids, jj2])
                        v1 = plsc.load_gather(src, [d_ids, jj2 + 1])
                        plsc.store_scatter(out_v, [row_ids, d_ids], v0)
                        plsc.store_scatter(out_v, [row_ids, d_ids + 64], v1)
                return carry

            lax.fori_loop(0, n_rows // 16, blk, 0)

        def slab_step(i, carry):
            s = wid + NW * i

            @pl.when(s < NSLAB)
            def _():
                pltpu.sync_copy(tableT_hbm.at[:, pl.ds(s * W, W)], slab_v)
                transpose_blocks(slab_v, W // 2)
                pltpu.sync_copy(out_v, tableP_hbm.at[pl.ds(s * (W // 2),
                                                           W // 2), :])

            return carry

        lax.fori_loop(0, NSLAB // NW + 1, slab_step, 0)

        @pl.when(wid == 0)
        def _():
            pltpu.sync_copy(tableT_hbm.at[:, pl.ds(TAIL0, EMBED_DIM)], tail_v)
            transpose_blocks(tail_v, EMBED_DIM // 2)
            pltpu.sync_copy(out_v.at[pl.ds(0, EMBED_DIM // 2), :],
                            tableP_hbm.at[pl.ds(TAIL0 // 2,
                                                EMBED_DIM // 2), :])

    return body(tableT)


def _gather(tableP, nodesT):
    mesh = plsc.VectorSubcoreMesh(core_axis_name="c", subcore_axis_name="s")

    @functools.partial(
        pl.kernel,
        mesh=mesh,
        out_type=jax.ShapeDtypeStruct((HIST, EMBED_DIM, BATCH), jnp.float32),
        scratch_types=[
            pltpu.VMEM((C,), jnp.int32),
            pltpu.VMEM((C,), jnp.int32),
            pltpu.VMEM((C,), jnp.int32),
            pltpu.VMEM((C,), jnp.int32),
            pltpu.VMEM((C,), jnp.int32),
            pltpu.VMEM((C, 2 * EMBED_DIM), jnp.float32),
            pltpu.VMEM((C, 2 * EMBED_DIM), jnp.float32),
            pltpu.VMEM((EMBED_DIM, C), jnp.float32),
            pltpu.SemaphoreType.DMA,
            pltpu.SemaphoreType.DMA,
            pltpu.SemaphoreType.DMA,
        ],
        compiler_params=pltpu.CompilerParams(use_tc_tiling_on_sc=True,
                                             needs_layout_passes=False),
    )
    def body(tableP_hbm, nodesT_hbm, outT_hbm, idx_raw, j0, j1, c0, c1,
             rows0, rows1, outT_v, sem_g0, sem_g1, sem_o):
        wid = lax.axis_index("s") * NC + lax.axis_index("c")
        bbase = wid * B_PER_W
        j_ref = (j0, j1)
        c_ref = (c0, c1)
        rows = (rows0, rows1)
        sem_g = (sem_g0, sem_g1)
        lane = lax.iota(jnp.int32, 16)

        def prep_and_fire(t, b):
            h = t // QC
            b0 = bbase + (t % QC) * C
            pltpu.sync_copy(nodesT_hbm.at[h, pl.ds(b0, C)], idx_raw)

            def vec_step(i, carry):
                v = idx_raw[pl.ds(i * 16, 16)]
                j_ref[b][pl.ds(i * 16, 16)] = lax.shift_right_logical(v, 1)
                c_ref[b][pl.ds(i * 16, 16)] = (v & 1) * EMBED_DIM
                return carry

            lax.fori_loop(0, C // 16, vec_step, 0)
            pltpu.async_copy(tableP_hbm.at[j_ref[b]], rows[b], sem_g[b])

        def gather_wait(b):
            pltpu.make_async_copy(tableP_hbm.at[j_ref[b]], rows[b],
                                  sem_g[b]).wait()

        def select_t(b):
            def sel_step(g, carry):
                row_ids = g * 16 + lane
                offs = c_ref[b][pl.ds(g * 16, 16)]
                for k in range(16):
                    dk = (lane + k) & 15
                    col = offs + dk
                    vals = []
                    for d0 in range(0, EMBED_DIM, 16):
                        vals.append(
                            plsc.load_gather(rows[b], [row_ids, col + d0]))
                    for q, v in enumerate(vals):
                        plsc.store_scatter(outT_v, [q * 16 + dk, row_ids], v)
                return carry

            lax.fori_loop(0, C // 16, sel_step, 0)

        def out_window(t):
            h = t // QC
            b0 = bbase + (t % QC) * C
            return outT_hbm.at[h, :, pl.ds(b0, C)]

        def out_start(t):
            pltpu.async_copy(outT_v, out_window(t), sem_o)

        def out_wait(t):
            pltpu.make_async_copy(outT_v, out_window(t), sem_o).wait()

        prep_and_fire(0, 0)

        def pair_step(t2, carry):
            t = 2 * t2
            prep_and_fire(t + 1, 1)
            gather_wait(0)

            @pl.when(t2 > 0)
            def _():
                out_wait(t - 1)

            select_t(0)
            out_start(t)
            @pl.when(t2 < T_STEPS // 2 - 1)
            def _():
                prep_and_fire(t + 2, 0)

            gather_wait(1)
            out_wait(t)
            select_t(1)
            out_start(t + 1)
            return carry

        lax.fori_loop(0, T_STEPS // 2, pair_step, 0)
        out_wait(T_STEPS - 1)

    return body(tableP, nodesT)


def kernel(nodes, table):
    tableP = _repack(table.T)
    nodesT = nodes.T.astype(jnp.int32)
    outT = _gather(tableP, nodesT)
    return outT.transpose(2, 0, 1)


# speedup vs baseline: 1.2632x; 1.2632x over previous
import functools

import jax
import jax.numpy as jnp
from jax import lax
from jax.experimental import pallas as pl
from jax.experimental.pallas import tpu as pltpu
from jax.experimental.pallas import tpu_sc as plsc

NUM_NODES = 1_000_000
EMBED_DIM = 64
BATCH = 16384
HIST = 50

_info = plsc.get_sparse_core_info()
NC, NS = _info.num_cores, _info.num_subcores
NW = NC * NS
B_PER_W = BATCH // NW
C = 256
QC = B_PER_W // C
T_STEPS = HIST * QC


def _gather(tableP, nodesT):
    mesh = plsc.VectorSubcoreMesh(core_axis_name="c", subcore_axis_name="s")

    @functools.partial(
        pl.kernel,
        mesh=mesh,
        out_type=jax.ShapeDtypeStruct((HIST, EMBED_DIM, BATCH), jnp.float32),
        scratch_types=[
            pltpu.VMEM((C,), jnp.int32),
            pltpu.VMEM((C,), jnp.int32),
            pltpu.VMEM((C,), jnp.int32),
            pltpu.VMEM((C,), jnp.int32),
            pltpu.VMEM((C,), jnp.int32),
            pltpu.VMEM((C, 2 * EMBED_DIM), jnp.float32),
            pltpu.VMEM((C, 2 * EMBED_DIM), jnp.float32),
            pltpu.VMEM((EMBED_DIM, C), jnp.float32),
            pltpu.SemaphoreType.DMA,
            pltpu.SemaphoreType.DMA,
            pltpu.SemaphoreType.DMA,
        ],
        compiler_params=pltpu.CompilerParams(use_tc_tiling_on_sc=True,
                                             needs_layout_passes=False),
    )
    def body(tableP_hbm, nodesT_hbm, outT_hbm, idx_raw, j0, j1, c0, c1,
             rows0, rows1, outT_v, sem_g0, sem_g1, sem_o):
        wid = lax.axis_index("s") * NC + lax.axis_index("c")
        bbase = wid * B_PER_W
        j_ref = (j0, j1)
        c_ref = (c0, c1)
        rows = (rows0, rows1)
        sem_g = (sem_g0, sem_g1)
        lane = lax.iota(jnp.int32, 16)

        def prep_and_fire(t, b):
            h = t // QC
            b0 = bbase + (t % QC) * C
            pltpu.sync_copy(nodesT_hbm.at[h, pl.ds(b0, C)], idx_raw)

            def vec_step(i, carry):
                v = idx_raw[pl.ds(i * 16, 16)]
                j_ref[b][pl.ds(i * 16, 16)] = lax.shift_right_logical(v, 1)
                c_ref[b][pl.ds(i * 16, 16)] = (v & 1) * EMBED_DIM
                return carry

            lax.fori_loop(0, C // 16, vec_step, 0)
            pltpu.async_copy(tableP_hbm.at[j_ref[b]], rows[b], sem_g[b])

        def gather_wait(b):
            pltpu.make_async_copy(tableP_hbm.at[j_ref[b]], rows[b],
                                  sem_g[b]).wait()

        def select_t(b):
            def sel_step(g2, carry):
                blocks = []
                for u in range(2):
                    g = 2 * g2 + u
                    blocks.append((g * 16 + lane,
                                   c_ref[b][pl.ds(g * 16, 16)]))
                for k in range(16):
                    dk = (lane + k) & 15
                    vals = []
                    for row_ids, offs in blocks:
                        col = offs + dk
                        for d0 in range(0, EMBED_DIM, 16):
                            vals.append(
                                (row_ids, d0,
                                 plsc.load_gather(rows[b],
                                                  [row_ids, col + d0])))
                    for row_ids, d0, v in vals:
                        plsc.store_scatter(outT_v, [d0 + dk, row_ids], v)
                return carry

            lax.fori_loop(0, C // 32, sel_step, 0)

        def out_window(t):
            h = t // QC
            b0 = bbase + (t % QC) * C
            return outT_hbm.at[h, :, pl.ds(b0, C)]

        def out_start(t):
            pltpu.async_copy(outT_v, out_window(t), sem_o)

        def out_wait(t):
            pltpu.make_async_copy(outT_v, out_window(t), sem_o).wait()

        prep_and_fire(0, 0)

        def pair_step(t2, carry):
            t = 2 * t2
            prep_and_fire(t + 1, 1)
            gather_wait(0)

            @pl.when(t2 > 0)
            def _():
                out_wait(t - 1)

            select_t(0)
            out_start(t)
            @pl.when(t2 < T_STEPS // 2 - 1)
            def _():
                prep_and_fire(t + 2, 0)

            gather_wait(1)
            out_wait(t)
            select_t(1)
            out_start(t + 1)
            return carry

        lax.fori_loop(0, T_STEPS // 2, pair_step, 0)
        out_wait(T_STEPS - 1)

    return body(tableP, nodesT)


def kernel(nodes, table):
    tableP = table.reshape(NUM_NODES // 2, 2 * EMBED_DIM)
    nodesT = nodes.T.astype(jnp.int32)
    outT = _gather(tableP, nodesT)
    return outT.transpose(2, 0, 1)


# speedup vs baseline: 1.2831x; 1.0158x over previous
import functools

import jax
import jax.numpy as jnp
from jax import lax
from jax.experimental import pallas as pl
from jax.experimental.pallas import tpu as pltpu
from jax.experimental.pallas import tpu_sc as plsc

NUM_NODES = 1_000_000
EMBED_DIM = 64
BATCH = 16384
HIST = 50

_info = plsc.get_sparse_core_info()
NC, NS = _info.num_cores, _info.num_subcores
NW = NC * NS
B_PER_W = BATCH // NW
C = 256
QC = B_PER_W // C
T_STEPS = HIST * QC


def _gather(tableP, nodesT):
    mesh = plsc.VectorSubcoreMesh(core_axis_name="c", subcore_axis_name="s")

    @functools.partial(
        pl.kernel,
        mesh=mesh,
        out_type=jax.ShapeDtypeStruct((HIST, EMBED_DIM, BATCH), jnp.float32),
        scratch_types=[
            pltpu.VMEM((C,), jnp.int32),
            pltpu.VMEM((C,), jnp.int32),
            pltpu.VMEM((C,), jnp.int32),
            pltpu.VMEM((C,), jnp.int32),
            pltpu.VMEM((C,), jnp.int32),
            pltpu.VMEM((C, 2 * EMBED_DIM), jnp.float32),
            pltpu.VMEM((C, 2 * EMBED_DIM), jnp.float32),
            pltpu.VMEM((EMBED_DIM, C), jnp.float32),
            pltpu.SemaphoreType.DMA,
            pltpu.SemaphoreType.DMA,
            pltpu.SemaphoreType.DMA,
        ],
        compiler_params=pltpu.CompilerParams(use_tc_tiling_on_sc=True,
                                             needs_layout_passes=False),
    )
    def body(tableP_hbm, nodesT_hbm, outT_hbm, idx_raw, j0, j1, c0, c1,
             rows0, rows1, outT_v, sem_g0, sem_g1, sem_o):
        wid = lax.axis_index("s") * NC + lax.axis_index("c")
        bbase = wid * B_PER_W
        j_ref = (j0, j1)
        c_ref = (c0, c1)
        rows = (rows0, rows1)
        sem_g = (sem_g0, sem_g1)
        lane = lax.iota(jnp.int32, 16)

        def prep_and_fire(t, b):
            h = t // QC
            b0 = bbase + (t % QC) * C
            pltpu.sync_copy(nodesT_hbm.at[h, pl.ds(b0, C)], idx_raw)

            def vec_step(i, carry):
                v = idx_raw[pl.ds(i * 16, 16)]
                j_ref[b][pl.ds(i * 16, 16)] = lax.shift_right_logical(v, 1)
                c_ref[b][pl.ds(i * 16, 16)] = (v & 1) * EMBED_DIM
                return carry

            lax.fori_loop(0, C // 16, vec_step, 0)
            pltpu.async_copy(tableP_hbm.at[j_ref[b]], rows[b], sem_g[b])

        def gather_wait(b):
            pltpu.make_async_copy(tableP_hbm.at[j_ref[b]], rows[b],
                                  sem_g[b]).wait()

        def select_t(b):
            def sel_step(g2, carry):
                blocks = []
                for u in range(4):
                    g = 4 * g2 + u
                    blocks.append((g * 16 + lane,
                                   c_ref[b][pl.ds(g * 16, 16)]))
                for k in range(16):
                    dk = (lane + k) & 15
                    vals = []
                    for row_ids, offs in blocks:
                        col = offs + dk
                        for d0 in range(0, EMBED_DIM, 16):
                            vals.append(
                                (row_ids, d0,
                                 plsc.load_gather(rows[b],
                                                  [row_ids, col + d0])))
                    for row_ids, d0, v in vals:
                        plsc.store_scatter(outT_v, [d0 + dk, row_ids], v)
                return carry

            lax.fori_loop(0, C // 64, sel_step, 0)

        def out_window(t):
            h = t // QC
            b0 = bbase + (t % QC) * C
            return outT_hbm.at[h, :, pl.ds(b0, C)]

        def out_start(t):
            pltpu.async_copy(outT_v, out_window(t), sem_o)

        def out_wait(t):
            pltpu.make_async_copy(outT_v, out_window(t), sem_o).wait()

        prep_and_fire(0, 0)

        def pair_step(t2, carry):
            t = 2 * t2
            prep_and_fire(t + 1, 1)
            gather_wait(0)

            @pl.when(t2 > 0)
            def _():
                out_wait(t - 1)

            select_t(0)
            out_start(t)
            @pl.when(t2 < T_STEPS // 2 - 1)
            def _():
                prep_and_fire(t + 2, 0)

            gather_wait(1)
            out_wait(t)
            select_t(1)
            out_start(t + 1)
            return carry

        lax.fori_loop(0, T_STEPS // 2, pair_step, 0)
        out_wait(T_STEPS - 1)

    return body(tableP, nodesT)


def kernel(nodes, table):
    tableP = table.reshape(NUM_NODES // 2, 2 * EMBED_DIM)
    nodesT = nodes.T.astype(jnp.int32)
    outT = _gather(tableP, nodesT)
    return outT.transpose(2, 0, 1)
